# Initial kernel scaffold; baseline (speedup 1.0000x reference)
import jax
import jax.numpy as jnp
from jax.experimental import pallas as pl


def kernel(input_ids, table):
    raise NotImplementedError("write your pallas kernel here")


# speedup vs baseline: 2.9733x; 2.9733x over previous
import functools

import jax
import jax.numpy as jnp
from jax import lax
from jax.experimental import pallas as pl
from jax.experimental.pallas import tpu as pltpu
from jax.experimental.pallas import tpu_sc as plsc

D = 128
C = 128


@functools.partial(jax.jit, static_argnames=("n_chunks", "nc", "ns"))
def _gather_sc(ids, table, n_chunks, nc, ns):
    nw = nc * ns
    mesh = plsc.VectorSubcoreMesh(core_axis_name="c", subcore_axis_name="s")

    @functools.partial(
        pl.kernel,
        mesh=mesh,
        out_type=jax.ShapeDtypeStruct((nw, n_chunks, C, D), jnp.float32),
        scratch_types=[
            pltpu.VMEM((n_chunks, C), jnp.int32),
            pltpu.VMEM((C, D), jnp.float32),
            pltpu.SemaphoreType.DMA,
        ],
    )
    def k(ids_hbm, table_hbm, out_hbm, idx_v, rows_v, sem):
        wid = lax.axis_index("s") * nc + lax.axis_index("c")
        pltpu.sync_copy(ids_hbm.at[wid], idx_v)

        def body(j, carry):
            pltpu.async_copy(table_hbm.at[idx_v.at[j]], rows_v, sem).wait()
            pltpu.sync_copy(rows_v, out_hbm.at[wid, j])
            return carry

        lax.fori_loop(0, n_chunks, body, 0)

    return k(ids, table)


def kernel(input_ids, table):
    b0, s = input_ids.shape
    b = b0 * s
    info = plsc.get_sparse_core_info()
    nc, ns = info.num_cores, info.num_subcores
    nw = nc * ns
    n_chunks = b // (nw * C)
    ids = input_ids.reshape(nw, n_chunks, C).astype(jnp.int32)
    out = _gather_sc(ids, table, n_chunks, nc, ns)
    return out.reshape(b0, s, D)


# speedup vs baseline: 3.3490x; 1.1264x over previous
import functools

import jax
import jax.numpy as jnp
from jax import lax
from jax.experimental import pallas as pl
from jax.experimental.pallas import tpu as pltpu
from jax.experimental.pallas import tpu_sc as plsc

D = 128
C = 128
NBUF = 5
PRE = 3


@functools.partial(jax.jit, static_argnames=("n_chunks", "nc", "ns"))
def _gather_sc(ids, table, n_chunks, nc, ns):
    nw = nc * ns
    mesh = plsc.VectorSubcoreMesh(core_axis_name="c", subcore_axis_name="s")

    @functools.partial(
        pl.kernel,
        mesh=mesh,
        out_type=jax.ShapeDtypeStruct((nw, n_chunks, C, D), jnp.float32),
        scratch_types=(
            [pltpu.VMEM((n_chunks, C), jnp.int32),
             pltpu.VMEM((NBUF, C, D), jnp.float32)]
            + [pltpu.SemaphoreType.DMA] * (2 * NBUF)
        ),
    )
    def k(ids_hbm, table_hbm, out_hbm, idx_v, rows_v, *sems):
        gsems, ssems = sems[:NBUF], sems[NBUF:]
        wid = lax.axis_index("s") * nc + lax.axis_index("c")
        pltpu.sync_copy(ids_hbm.at[wid], idx_v)

        def start_gather(j, b):
            pltpu.async_copy(table_hbm.at[idx_v.at[j]], rows_v.at[b], gsems[b])

        def wait_gather(b):
            pltpu.make_async_copy(
                table_hbm.at[idx_v.at[0]], rows_v.at[b], gsems[b]).wait()

        def start_store(j, b):
            pltpu.async_copy(rows_v.at[b], out_hbm.at[wid, j], ssems[b])

        def wait_store(b):
            pltpu.make_async_copy(
                rows_v.at[b], out_hbm.at[wid, 0], ssems[b]).wait()

        for j in range(PRE):
            start_gather(j, j % NBUF)

        def outer(i, carry):
            g = i * NBUF
            for b in range(NBUF):
                j = g + b
                wait_gather(b)
                start_store(j, b)
                jn = j + PRE
                bn = (b + PRE) % NBUF

                @pl.when(jn < n_chunks)
                def _():
                    @pl.when(jn >= NBUF)
                    def _():
                        wait_store(bn)
                    start_gather(jn, bn)

            return carry

        lax.fori_loop(0, n_chunks // NBUF, outer, 0)
        for b in range(NBUF):
            wait_store(b)

    return k(ids, table)


def kernel(input_ids, table):
    b0, s = input_ids.shape
    b = b0 * s
    info = plsc.get_sparse_core_info()
    nc, ns = info.num_cores, info.num_subcores
    nw = nc * ns
    n_chunks = b // (nw * C)
    ids = input_ids.reshape(nw, n_chunks, C).astype(jnp.int32)
    out = _gather_sc(ids, table, n_chunks, nc, ns)
    return out.reshape(b0, s, D)


# speedup vs baseline: 5.9727x; 1.7835x over previous
import functools

import jax
import jax.numpy as jnp
from jax import lax
from jax.experimental import pallas as pl
from jax.experimental.pallas import tpu as pltpu
from jax.experimental.pallas import tpu_sc as plsc

D = 128
NBUF = 8
PRE = 5


@functools.partial(jax.jit, static_argnames=("s", "nc", "ns"))
def _gather_sc(ids, table, s, nc, ns):
    nw = nc * ns
    n_seq = ids.shape[0]
    per_w = n_seq // nw
    mesh = plsc.VectorSubcoreMesh(core_axis_name="c", subcore_axis_name="s")

    @functools.partial(
        pl.kernel,
        mesh=mesh,
        out_type=jax.ShapeDtypeStruct((n_seq, s, D), jnp.float32),
        scratch_types=(
            [pltpu.VMEM((per_w, s), jnp.int32),
             pltpu.VMEM((NBUF, s, D), jnp.float32)]
            + [pltpu.SemaphoreType.DMA] * (2 * NBUF)
        ),
    )
    def k(ids_hbm, table_hbm, out_hbm, idx_v, rows_v, *sems):
        gsems, ssems = sems[:NBUF], sems[NBUF:]
        wid = lax.axis_index("s") * nc + lax.axis_index("c")
        base = wid * per_w
        pltpu.sync_copy(ids_hbm.at[pl.ds(base, per_w)], idx_v)

        def start_gather(j, b):
            pltpu.async_copy(table_hbm.at[idx_v.at[j]], rows_v.at[b], gsems[b])

        def wait_gather(b):
            pltpu.make_async_copy(
                table_hbm.at[idx_v.at[0]], rows_v.at[b], gsems[b]).wait()

        def start_store(j, b):
            pltpu.async_copy(rows_v.at[b], out_hbm.at[base + j], ssems[b])

        def wait_store(b):
            pltpu.make_async_copy(
                rows_v.at[b], out_hbm.at[base], ssems[b]).wait()

        for j in range(PRE):
            start_gather(j, j % NBUF)

        def outer(i, carry):
            g = i * NBUF
            for b in range(NBUF):
                j = g + b
                wait_gather(b)
                start_store(j, b)
                jn = j + PRE
                bn = (b + PRE) % NBUF

                @pl.when(jn < per_w)
                def _():
                    @pl.when(jn >= NBUF)
                    def _():
                        wait_store(bn)
                    start_gather(jn, bn)

            return carry

        lax.fori_loop(0, per_w // NBUF, outer, 0)
        for b in range(NBUF):
            wait_store(b)

    return k(ids, table)


def kernel(input_ids, table):
    b0, s = input_ids.shape
    info = plsc.get_sparse_core_info()
    nc, ns = info.num_cores, info.num_subcores
    ids = input_ids.astype(jnp.int32)
    return _gather_sc(ids, table, s, nc, ns)
